# Initial kernel scaffold; baseline (speedup 1.0000x reference)
import jax
import jax.numpy as jnp
from jax.experimental import pallas as pl


def kernel(x, adj, W_embed, b_embed, W_g1, b_g1, W_g2, b_g2, W_m1, b_m1, g1, be1, W_m2, b_m2, g2, be2, W_m3, b_m3):
    raise NotImplementedError("write your pallas kernel here")


# speedup vs baseline: 9.8768x; 9.8768x over previous
import functools

import jax
import jax.numpy as jnp
from jax import lax
from jax.experimental import pallas as pl
from jax.experimental.pallas import tpu as pltpu
from jax.experimental.pallas import tpu_sc as plsc

NC, NS = 2, 16
NW = NC * NS
CHUNK = 128
BLK = 512


def _mesh():
    return plsc.VectorSubcoreMesh(core_axis_name="c", subcore_axis_name="s")


def _sc_hist(dst_pad, ones128, zeros128, npad):
    epad = dst_pad.shape[0]
    ept = epad // NW
    nch = ept // CHUNK
    rps = npad // NS
    w = ones128.shape[1]

    @functools.partial(
        pl.kernel, mesh=_mesh(),
        out_type=jax.ShapeDtypeStruct((NC, npad, w), jnp.float32),
        scratch_types=[pltpu.VMEM((CHUNK,), jnp.int32),
                       pltpu.VMEM((CHUNK, w), jnp.float32),
                       pltpu.VMEM_SHARED((npad, w), jnp.float32)],
    )
    def hist_kernel(dst_hbm, ones_hbm, zeros_hbm, o_hbm, dstv, onesv, acc_sh):
        c = lax.axis_index("c")
        s = lax.axis_index("s")
        wid = s * NC + c
        pltpu.sync_copy(zeros_hbm.at[pl.ds(s * rps, rps)],
                        acc_sh.at[pl.ds(s * rps, rps)])
        pltpu.sync_copy(ones_hbm, onesv)
        plsc.subcore_barrier()
        base = wid * ept

        @pl.loop(0, nch)
        def _(ci):
            pltpu.sync_copy(dst_hbm.at[pl.ds(base + ci * CHUNK, CHUNK)], dstv)
            pltpu.sync_copy(onesv, acc_sh.at[dstv], add=True)

        plsc.subcore_barrier()
        pltpu.sync_copy(acc_sh.at[pl.ds(s * rps, rps)],
                        o_hbm.at[c].at[pl.ds(s * rps, rps)])

    return hist_kernel(dst_pad, ones128, zeros128)


def _sc_conv(hp, src_pad, dst_pad, zeros128, npad):
    epad = src_pad.shape[0]
    ept = epad // NW
    nch = ept // CHUNK
    rps = npad // NS
    h = hp.shape[1]

    @functools.partial(
        pl.kernel, mesh=_mesh(),
        out_type=jax.ShapeDtypeStruct((NC, npad, h), jnp.float32),
        scratch_types=[pltpu.VMEM((CHUNK,), jnp.int32),
                       pltpu.VMEM((CHUNK,), jnp.int32),
                       pltpu.VMEM((CHUNK, h), jnp.float32),
                       pltpu.VMEM_SHARED((npad, h), jnp.float32),
                       pltpu.SemaphoreType.DMA],
    )
    def conv_kernel(hp_hbm, src_hbm, dst_hbm, zeros_hbm, o_hbm,
                    srcv, dstv, rows, acc_sh, sem):
        c = lax.axis_index("c")
        s = lax.axis_index("s")
        wid = s * NC + c
        pltpu.sync_copy(zeros_hbm.at[pl.ds(s * rps, rps)],
                        acc_sh.at[pl.ds(s * rps, rps)])
        plsc.subcore_barrier()
        base = wid * ept

        @pl.loop(0, nch)
        def _(ci):
            off = base + ci * CHUNK
            pltpu.sync_copy(src_hbm.at[pl.ds(off, CHUNK)], srcv)
            pltpu.sync_copy(dst_hbm.at[pl.ds(off, CHUNK)], dstv)
            pltpu.async_copy(hp_hbm.at[srcv], rows, sem).wait()
            pltpu.sync_copy(rows, acc_sh.at[dstv], add=True)

        plsc.subcore_barrier()
        pltpu.sync_copy(acc_sh.at[pl.ds(s * rps, rps)],
                        o_hbm.at[c].at[pl.ds(s * rps, rps)])

    return conv_kernel(hp, src_pad, dst_pad, zeros128)


def _embed_body(x_ref, w_ref, b_ref, o_ref):
    o_ref[...] = (jnp.dot(x_ref[...], w_ref[...],
                          preferred_element_type=jnp.float32) + b_ref[...])


def _embed(x, W, b):
    npad, d = x.shape
    h = W.shape[1]
    return pl.pallas_call(
        _embed_body,
        grid=(npad // BLK,),
        in_specs=[pl.BlockSpec((BLK, d), lambda i: (i, 0)),
                  pl.BlockSpec((d, h), lambda i: (0, 0)),
                  pl.BlockSpec((1, h), lambda i: (0, 0))],
        out_specs=pl.BlockSpec((BLK, h), lambda i: (i, 0)),
        out_shape=jax.ShapeDtypeStruct((npad, h), jnp.float32),
    )(x, W, b.reshape(1, h))


def _prescale_body(h_ref, w_ref, ha_ref, hb_ref, hp_ref, dinv_ref):
    deg = ha_ref[:, 0:1] + hb_ref[:, 0:1] + 1.0
    dinv = 1.0 / jnp.sqrt(deg)
    hw = jnp.dot(h_ref[...], w_ref[...], preferred_element_type=jnp.float32)
    hp_ref[...] = hw * dinv
    dinv_ref[...] = dinv


def _prescale(h0, W, ha, hb):
    npad, h = h0.shape
    return pl.pallas_call(
        _prescale_body,
        grid=(npad // BLK,),
        in_specs=[pl.BlockSpec((BLK, h), lambda i: (i, 0)),
                  pl.BlockSpec((h, h), lambda i: (0, 0)),
                  pl.BlockSpec((BLK, 128), lambda i: (i, 0)),
                  pl.BlockSpec((BLK, 128), lambda i: (i, 0))],
        out_specs=[pl.BlockSpec((BLK, h), lambda i: (i, 0)),
                   pl.BlockSpec((BLK, 1), lambda i: (i, 0))],
        out_shape=[jax.ShapeDtypeStruct((npad, h), jnp.float32),
                   jax.ShapeDtypeStruct((npad, 1), jnp.float32)],
    )(h0, W, ha, hb)


def _conv_next_body(aa_ref, ab_ref, hp_ref, dv_ref, b_ref, w_ref, o_ref):
    dv = dv_ref[...]
    s = (aa_ref[...] + ab_ref[...] + hp_ref[...]) * dv + b_ref[...]
    s = jnp.maximum(s, 0.0)
    o_ref[...] = jnp.dot(s, w_ref[...],
                         preferred_element_type=jnp.float32) * dv


def _conv_next(aa, ab, hp, dinv, b, Wn):
    npad, h = hp.shape
    return pl.pallas_call(
        _conv_next_body,
        grid=(npad // BLK,),
        in_specs=[pl.BlockSpec((BLK, h), lambda i: (i, 0)),
                  pl.BlockSpec((BLK, h), lambda i: (i, 0)),
                  pl.BlockSpec((BLK, h), lambda i: (i, 0)),
                  pl.BlockSpec((BLK, 1), lambda i: (i, 0)),
                  pl.BlockSpec((1, h), lambda i: (0, 0)),
                  pl.BlockSpec((h, h), lambda i: (0, 0))],
        out_specs=pl.BlockSpec((BLK, h), lambda i: (i, 0)),
        out_shape=jax.ShapeDtypeStruct((npad, h), jnp.float32),
    )(aa, ab, hp, dinv, b.reshape(1, h), Wn)


def _layer_norm(t, g, b):
    mu = jnp.mean(t, axis=-1, keepdims=True)
    var = jnp.mean((t - mu) ** 2, axis=-1, keepdims=True)
    return (t - mu) / jnp.sqrt(var + 1e-5) * g + b


def _head_body(aa_ref, ab_ref, hp_ref, dv_ref, bg_ref, wm1_ref, bm1_ref,
               g1_ref, be1_ref, wm2_ref, bm2_ref, g2_ref, be2_ref,
               wm3_ref, bm3_ref, o_ref):
    dv = dv_ref[...]
    t = (aa_ref[...] + ab_ref[...] + hp_ref[...]) * dv + bg_ref[...]
    t = jnp.maximum(t, 0.0)
    t = jnp.dot(t, wm1_ref[...], preferred_element_type=jnp.float32) + bm1_ref[...]
    t = jnp.maximum(_layer_norm(t, g1_ref[...], be1_ref[...]), 0.0)
    t = jnp.dot(t, wm2_ref[...], preferred_element_type=jnp.float32) + bm2_ref[...]
    t = jnp.maximum(_layer_norm(t, g2_ref[...], be2_ref[...]), 0.0)
    o_ref[...] = jnp.dot(t, wm3_ref[...],
                         preferred_element_type=jnp.float32) + bm3_ref[...]


def _head(aa, ab, hp, dinv, b_g2, W_m1, b_m1, g1, be1, W_m2, b_m2, g2, be2,
          W_m3, b_m3):
    npad, h = hp.shape
    row = lambda i: (i, 0)
    fixed = lambda i: (0, 0)
    return pl.pallas_call(
        _head_body,
        grid=(npad // BLK,),
        in_specs=[pl.BlockSpec((BLK, h), row),
                  pl.BlockSpec((BLK, h), row),
                  pl.BlockSpec((BLK, h), row),
                  pl.BlockSpec((BLK, 1), row),
                  pl.BlockSpec((1, h), fixed),
                  pl.BlockSpec((h, h), fixed),
                  pl.BlockSpec((1, h), fixed),
                  pl.BlockSpec((1, h), fixed),
                  pl.BlockSpec((1, h), fixed),
                  pl.BlockSpec((h, h), fixed),
                  pl.BlockSpec((1, h), fixed),
                  pl.BlockSpec((1, h), fixed),
                  pl.BlockSpec((1, h), fixed),
                  pl.BlockSpec((h, h), fixed),
                  pl.BlockSpec((1, h), fixed)],
        out_specs=pl.BlockSpec((BLK, h), row),
        out_shape=jax.ShapeDtypeStruct((npad, h), jnp.float32),
    )(aa, ab, hp, dinv, b_g2.reshape(1, h), W_m1, b_m1.reshape(1, h),
      g1.reshape(1, h), be1.reshape(1, h), W_m2, b_m2.reshape(1, h),
      g2.reshape(1, h), be2.reshape(1, h), W_m3, b_m3.reshape(1, h))


def kernel(x, adj, W_embed, b_embed, W_g1, b_g1, W_g2, b_g2,
           W_m1, b_m1, g1, be1, W_m2, b_m2, g2, be2, W_m3, b_m3):
    n, d = x.shape
    e = adj.shape[1]
    npad = -(-(n + 1) // BLK) * BLK
    step = NW * CHUNK
    epad = -(-e // step) * step

    pad_idx = jnp.full((epad - e,), n, jnp.int32)
    src = jnp.concatenate([adj[0].astype(jnp.int32), pad_idx])
    dst = jnp.concatenate([adj[1].astype(jnp.int32), pad_idx])
    xp = jnp.concatenate([x, jnp.zeros((npad - n, d), jnp.float32)], axis=0)

    ones128 = jnp.ones((CHUNK, 128), jnp.float32)
    zeros128 = jnp.zeros((npad, W_g1.shape[1]), jnp.float32)

    hist = _sc_hist(dst, ones128, zeros128, npad)
    h0 = _embed(xp, W_embed, b_embed)
    h1p, dinv = _prescale(h0, W_g1, hist[0], hist[1])
    acc = _sc_conv(h1p, src, dst, zeros128, npad)
    h2p = _conv_next(acc[0], acc[1], h1p, dinv, b_g1, W_g2)
    acc2 = _sc_conv(h2p, src, dst, zeros128, npad)
    out = _head(acc2[0], acc2[1], h2p, dinv, b_g2, W_m1, b_m1, g1, be1,
                W_m2, b_m2, g2, be2, W_m3, b_m3)
    return out[:n]
